# baseline (speedup 1.0000x reference)
import functools

import jax
import jax.numpy as jnp
from jax import lax
from jax.experimental import pallas as pl
from jax.experimental.pallas import tpu as pltpu
from jax.experimental.pallas import tpu_sc as plsc

VOCAB = 1000000
HIDDEN = 64
SCALE = 8.0
LANES = 16
NUM_CORES = 2
NUM_SUBCORES = 16
NW = NUM_CORES * NUM_SUBCORES
BATCH = 4096
SEQ = 200
BW = BATCH // NW
TW = 2 * HIDDEN
HT = HIDDEN // 8
CBLK = 256
NBLK = 999936 // CBLK
TAIL0 = NBLK * CBLK
TAILW = VOCAB - TAIL0
RSTEPS = -(-NBLK // NW)
RBUF = 2
GBUF = 4
GOUT = SEQ // GBUF
OROWS = SEQ * HIDDEN * BATCH // TW

_mesh = plsc.VectorSubcoreMesh(core_axis_name="c", subcore_axis_name="s")
_params = pltpu.CompilerParams(use_tc_tiling_on_sc=True, needs_layout_passes=False)


@functools.partial(
    pl.kernel,
    out_type=jax.ShapeDtypeStruct((VOCAB, TW), jnp.float32),
    mesh=_mesh,
    scratch_types=[
        pltpu.VMEM((RBUF, HT, 8, CBLK), jnp.float32),
        pltpu.VMEM((RBUF, CBLK, TW), jnp.float32),
        pltpu.VMEM((HIDDEN, TAILW), jnp.float32),
        pltpu.VMEM((TAILW, TW), jnp.float32),
        pltpu.SemaphoreType.DMA,
        pltpu.SemaphoreType.DMA,
        pltpu.SemaphoreType.DMA,
        pltpu.SemaphoreType.DMA,
    ],
    compiler_params=_params,
)
def _repack(wt_hbm, tab_hbm, in_v, out_v, tin_v, tout_v, si0, si1, so0, so1):
    wid = lax.axis_index("s") * NUM_CORES + lax.axis_index("c")
    sin = [si0, si1]
    sout = [so0, so1]
    ci = [lax.iota(jnp.int32, LANES) + g * LANES for g in range(CBLK // LANES)]
    scale = jnp.float32(SCALE)

    def in_cps(t, k):
        c0 = (wid + t * NW) * CBLK
        return [
            pltpu.make_async_copy(
                wt_hbm.at[pl.ds(8 * i, 8), pl.ds(c0, CBLK)],
                in_v.at[k, i],
                sin[k],
            )
            for i in range(HT)
        ]

    def out_cp(t, k):
        c0 = (wid + t * NW) * CBLK
        return pltpu.make_async_copy(
            out_v.at[k], tab_hbm.at[pl.ds(c0, CBLK)], sout[k]
        )

    for k in range(RBUF):
        @pl.when(wid + k * NW < NBLK)
        def _():
            for cp in in_cps(k, k):
                cp.start()

    def outer(T, carry):
        for k in range(RBUF):
            t = T * RBUF + k

            @pl.when(wid + t * NW < NBLK)
            def _():
                for cp in in_cps(t, k):
                    cp.wait()

                @pl.when(t >= RBUF)
                def _():
                    out_cp(t - RBUF, k).wait()

                def hrow(i, hc):
                    for r in range(8):
                        hv = jnp.full((LANES,), 8 * i + r, jnp.int32)
                        for g in range(CBLK // LANES):
                            seg = in_v[k, i, r, pl.ds(g * LANES, LANES)]
                            plsc.store_scatter(
                                out_v.at[k], [ci[g], hv], seg * scale
                            )
                    return hc

                lax.fori_loop(0, HT, hrow, 0)
                out_cp(t, k).start()
                tn = t + RBUF

                @pl.when(wid + tn * NW < NBLK)
                def _():
                    for cp in in_cps(tn, k):
                        cp.start()

        return carry

    lax.fori_loop(0, -(-RSTEPS // RBUF), outer, 0)
    for k in range(RBUF):
        t_last = RSTEPS - RBUF + k

        @pl.when(wid + t_last * NW < NBLK)
        def _():
            out_cp(t_last, k).wait()

    @pl.when(wid == NW - 1)
    def _tail():
        pltpu.sync_copy(wt_hbm.at[pl.ds(0, HIDDEN), pl.ds(TAIL0, TAILW)], tin_v)
        for h in range(HIDDEN):
            hv = jnp.full((LANES,), h, jnp.int32)
            for g in range(TAILW // LANES):
                seg = tin_v[h, pl.ds(g * LANES, LANES)]
                plsc.store_scatter(tout_v, [ci[g], hv], seg * scale)
        pltpu.sync_copy(tout_v, tab_hbm.at[pl.ds(TAIL0, TAILW)])


@functools.partial(
    pl.kernel,
    out_type=jax.ShapeDtypeStruct((OROWS, TW), jnp.float32),
    mesh=_mesh,
    scratch_types=[
        pltpu.VMEM((SEQ, BW), jnp.int32),
        pltpu.VMEM((GBUF, BW, TW), jnp.float32),
        pltpu.VMEM((GBUF, HIDDEN, BW), jnp.float32),
        pltpu.VMEM((GBUF, HIDDEN), jnp.int32),
        pltpu.SemaphoreType.DMA,
        pltpu.SemaphoreType.DMA,
        pltpu.SemaphoreType.DMA,
        pltpu.SemaphoreType.DMA,
        pltpu.SemaphoreType.DMA,
        pltpu.SemaphoreType.DMA,
        pltpu.SemaphoreType.DMA,
        pltpu.SemaphoreType.DMA,
    ],
    compiler_params=_params,
)
def _embed(
    xt_hbm, tab_hbm, out_hbm, idx_v, unit_v, ot_v, sidx_v,
    g0, g1, g2, g3, o0, o1, o2, o3,
):
    wid = lax.axis_index("s") * NUM_CORES + lax.axis_index("c")
    b0 = wid * BW
    gsem = [g0, g1, g2, g3]
    osem = [o0, o1, o2, o3]
    lane_iota = lax.iota(jnp.int32, LANES)
    zeros = jnp.zeros((LANES,), jnp.float32)
    hstep = BATCH // TW

    pltpu.sync_copy(xt_hbm.at[pl.ds(0, SEQ), pl.ds(b0, BW)], idx_v)

    def g_cp(s, k):
        return pltpu.make_async_copy(
            tab_hbm.at[idx_v.at[s]], unit_v.at[k], gsem[k]
        )

    def o_cp(k):
        return pltpu.make_async_copy(
            ot_v.at[k], out_hbm.at[sidx_v.at[k]], osem[k]
        )

    def set_sidx(s, k):
        base = s * (HIDDEN * hstep) + wid
        for g in range(HIDDEN // LANES):
            sidx_v[k, pl.ds(g * LANES, LANES)] = (
                base + (g * LANES + lane_iota) * hstep
            )

    for k in range(GBUF):
        g_cp(k, k).start()

    def outer(T, carry):
        for k in range(GBUF):
            s = T * GBUF + k
            g_cp(s, k).wait()

            @pl.when(T > 0)
            def _():
                o_cp(k).wait()

            set_sidx(s, k)

            def group(g, gc):
                for c in range(HIDDEN // LANES):
                    col = c * LANES + lane_iota
                    for j in range(LANES):
                        r = g * LANES + j
                        seg = unit_v[k, r, pl.ds(c * LANES, LANES)]
                        plsc.store_scatter(
                            ot_v.at[k],
                            [col, jnp.full((LANES,), r, jnp.int32)],
                            seg,
                        )
                return gc

            lax.fori_loop(0, BW // LANES, group, 0)

            def zfix(g, gc):
                iv = idx_v[s, pl.ds(g * LANES, LANES)]
                nz = plsc.all_reduce_population_count(iv == 0)

                @pl.when(nz[0] > 0)
                def _():
                    for j in range(LANES):
                        @pl.when(iv[j] == 0)
                        def _():
                            r = g * LANES + j
                            for c in range(HIDDEN // LANES):
                                plsc.store_scatter(
                                    ot_v.at[k],
                                    [
                                        c * LANES + lane_iota,
                                        jnp.full((LANES,), r, jnp.int32),
                                    ],
                                    zeros,
                                )
                return gc

            lax.fori_loop(0, BW // LANES, zfix, 0)
            o_cp(k).start()

            @pl.when(s + GBUF < SEQ)
            def _():
                g_cp(s + GBUF, k).start()

        return carry

    lax.fori_loop(0, GOUT, outer, 0)
    for k in range(GBUF):
        o_cp(k).wait()


def kernel(x, shared_weights):
    xt = x.astype(jnp.int32).T
    wt = shared_weights.T
    tab = _repack(wt)
    out_r = _embed(xt, tab)
    out_st = out_r.reshape(SEQ, HIDDEN, BATCH)
    return jnp.transpose(out_st, (2, 0, 1))


# speedup vs baseline: 1.9113x; 1.9113x over previous
import functools

import jax
import jax.numpy as jnp
from jax import lax
from jax.experimental import pallas as pl
from jax.experimental.pallas import tpu as pltpu
from jax.experimental.pallas import tpu_sc as plsc

VOCAB = 1000000
HIDDEN = 64
SCALE = 8.0
LANES = 16
NUM_CORES = 2
NUM_SUBCORES = 16
NW = NUM_CORES * NUM_SUBCORES
BATCH = 4096
SEQ = 200
BW = BATCH // NW
TW = 2 * HIDDEN
HT = HIDDEN // 8
CBLK = 256
NBLK = 999936 // CBLK
TAIL0 = NBLK * CBLK
TAILW = VOCAB - TAIL0
RSTEPS = -(-NBLK // NW)
RBUF = 2
GBUF = 4
GOUT = SEQ // GBUF
OROWS = SEQ * HIDDEN * BATCH // TW

_mesh = plsc.VectorSubcoreMesh(core_axis_name="c", subcore_axis_name="s")
_params = pltpu.CompilerParams(use_tc_tiling_on_sc=True, needs_layout_passes=False)


@functools.partial(
    pl.kernel,
    out_type=jax.ShapeDtypeStruct((VOCAB, TW), jnp.float32),
    mesh=_mesh,
    scratch_types=[
        pltpu.VMEM((RBUF, HT, 8, CBLK), jnp.float32),
        pltpu.VMEM((RBUF, CBLK, TW), jnp.float32),
        pltpu.VMEM((HIDDEN, TAILW), jnp.float32),
        pltpu.VMEM((TAILW, TW), jnp.float32),
        pltpu.SemaphoreType.DMA,
        pltpu.SemaphoreType.DMA,
        pltpu.SemaphoreType.DMA,
        pltpu.SemaphoreType.DMA,
    ],
    compiler_params=_params,
)
def _repack(wt_hbm, tab_hbm, in_v, out_v, tin_v, tout_v, si0, si1, so0, so1):
    wid = lax.axis_index("s") * NUM_CORES + lax.axis_index("c")
    sin = [si0, si1]
    sout = [so0, so1]
    lane_iota = lax.iota(jnp.int32, LANES)
    ci = [lane_iota + g * LANES for g in range(CBLK // LANES)]
    hvec = [lane_iota + h0 * LANES for h0 in range(HIDDEN // LANES)]
    hdiv = [lax.shift_right_logical(h, 3) for h in hvec]
    hmod = [jnp.bitwise_and(h, 7) for h in hvec]
    scale = jnp.float32(SCALE)

    def in_cps(t, k):
        c0 = (wid + t * NW) * CBLK
        return [
            pltpu.make_async_copy(
                wt_hbm.at[pl.ds(8 * i, 8), pl.ds(c0, CBLK)],
                in_v.at[k, i],
                sin[k],
            )
            for i in range(HT)
        ]

    def out_cp(t, k):
        c0 = (wid + t * NW) * CBLK
        return pltpu.make_async_copy(
            out_v.at[k], tab_hbm.at[pl.ds(c0, CBLK)], sout[k]
        )

    for k in range(RBUF):
        @pl.when(wid + k * NW < NBLK)
        def _():
            for cp in in_cps(k, k):
                cp.start()

    def outer(T, carry):
        for k in range(RBUF):
            t = T * RBUF + k

            @pl.when(wid + t * NW < NBLK)
            def _():
                for cp in in_cps(t, k):
                    cp.wait()

                @pl.when(t >= RBUF)
                def _():
                    out_cp(t - RBUF, k).wait()

                def diag(u0, hc):
                    uv = jnp.bitwise_and(u0 + lane_iota, CBLK - 1)
                    for hi in range(HIDDEN // LANES):
                        vv = plsc.load_gather(
                            in_v.at[k], [hdiv[hi], hmod[hi], uv]
                        )
                        plsc.store_scatter(
                            out_v.at[k], [uv, hvec[hi]], vv * scale
                        )
                    return hc

                lax.fori_loop(0, CBLK, diag, 0)
                out_cp(t, k).start()
                tn = t + RBUF

                @pl.when(wid + tn * NW < NBLK)
                def _():
                    for cp in in_cps(tn, k):
                        cp.start()

        return carry

    lax.fori_loop(0, -(-RSTEPS // RBUF), outer, 0)
    for k in range(RBUF):
        t_last = RSTEPS - RBUF + k

        @pl.when(wid + t_last * NW < NBLK)
        def _():
            out_cp(t_last, k).wait()

    @pl.when(wid == NW - 1)
    def _tail():
        pltpu.sync_copy(wt_hbm.at[pl.ds(0, HIDDEN), pl.ds(TAIL0, TAILW)], tin_v)
        for h in range(HIDDEN):
            hv = jnp.full((LANES,), h, jnp.int32)
            for g in range(TAILW // LANES):
                seg = tin_v[h, pl.ds(g * LANES, LANES)]
                plsc.store_scatter(tout_v, [ci[g], hv], seg * scale)
        pltpu.sync_copy(tout_v, tab_hbm.at[pl.ds(TAIL0, TAILW)])


@functools.partial(
    pl.kernel,
    out_type=jax.ShapeDtypeStruct((OROWS, TW), jnp.float32),
    mesh=_mesh,
    scratch_types=[
        pltpu.VMEM((SEQ, BW), jnp.int32),
        pltpu.VMEM((GBUF, BW, TW), jnp.float32),
        pltpu.VMEM((GBUF, HIDDEN, BW), jnp.float32),
        pltpu.VMEM((GBUF, HIDDEN), jnp.int32),
        pltpu.SemaphoreType.DMA,
        pltpu.SemaphoreType.DMA,
        pltpu.SemaphoreType.DMA,
        pltpu.SemaphoreType.DMA,
        pltpu.SemaphoreType.DMA,
        pltpu.SemaphoreType.DMA,
        pltpu.SemaphoreType.DMA,
        pltpu.SemaphoreType.DMA,
    ],
    compiler_params=_params,
)
def _embed(
    xt_hbm, tab_hbm, out_hbm, idx_v, unit_v, ot_v, sidx_v,
    g0, g1, g2, g3, o0, o1, o2, o3,
):
    wid = lax.axis_index("s") * NUM_CORES + lax.axis_index("c")
    b0 = wid * BW
    gsem = [g0, g1, g2, g3]
    osem = [o0, o1, o2, o3]
    lane_iota = lax.iota(jnp.int32, LANES)
    hvec = [lane_iota + h0 * LANES for h0 in range(HIDDEN // LANES)]
    zeros = jnp.zeros((LANES,), jnp.float32)
    hstep = BATCH // TW

    pltpu.sync_copy(xt_hbm.at[pl.ds(0, SEQ), pl.ds(b0, BW)], idx_v)

    def g_cp(s, k):
        return pltpu.make_async_copy(
            tab_hbm.at[idx_v.at[s]], unit_v.at[k], gsem[k]
        )

    def o_cp(k):
        return pltpu.make_async_copy(
            ot_v.at[k], out_hbm.at[sidx_v.at[k]], osem[k]
        )

    def set_sidx(s, k):
        base = s * (HIDDEN * hstep) + wid
        for g in range(HIDDEN // LANES):
            sidx_v[k, pl.ds(g * LANES, LANES)] = (
                base + (g * LANES + lane_iota) * hstep
            )

    for k in range(GBUF):
        g_cp(k, k).start()

    def outer(T, carry):
        for k in range(GBUF):
            s = T * GBUF + k
            g_cp(s, k).wait()

            @pl.when(T > 0)
            def _():
                o_cp(k).wait()

            set_sidx(s, k)

            def diag(r0, gc):
                rv = jnp.bitwise_and(r0 + lane_iota, BW - 1)
                for hi in range(HIDDEN // LANES):
                    vv = plsc.load_gather(unit_v.at[k], [rv, hvec[hi]])
                    plsc.store_scatter(ot_v.at[k], [hvec[hi], rv], vv)
                return gc

            lax.fori_loop(0, BW, diag, 0)

            def zfix(g, gc):
                iv = idx_v[s, pl.ds(g * LANES, LANES)]
                nz = plsc.all_reduce_population_count(iv == 0)

                @pl.when(nz[0] > 0)
                def _():
                    for j in range(LANES):
                        @pl.when(iv[j] == 0)
                        def _():
                            r = g * LANES + j
                            for c in range(HIDDEN // LANES):
                                plsc.store_scatter(
                                    ot_v.at[k],
                                    [
                                        c * LANES + lane_iota,
                                        jnp.full((LANES,), r, jnp.int32),
                                    ],
                                    zeros,
                                )
                return gc

            lax.fori_loop(0, BW // LANES, zfix, 0)
            o_cp(k).start()

            @pl.when(s + GBUF < SEQ)
            def _():
                g_cp(s + GBUF, k).start()

        return carry

    lax.fori_loop(0, GOUT, outer, 0)
    for k in range(GBUF):
        o_cp(k).wait()


def kernel(x, shared_weights):
    xt = x.astype(jnp.int32).T
    wt = shared_weights.T
    tab = _repack(wt)
    out_r = _embed(xt, tab)
    out_st = out_r.reshape(SEQ, HIDDEN, BATCH)
    return jnp.transpose(out_st, (2, 0, 1))


# speedup vs baseline: 1.9668x; 1.0290x over previous
import functools

import jax
import jax.numpy as jnp
from jax import lax
from jax.experimental import pallas as pl
from jax.experimental.pallas import tpu as pltpu
from jax.experimental.pallas import tpu_sc as plsc

VOCAB = 1000000
HIDDEN = 64
SCALE = 8.0
LANES = 16
NUM_CORES = 2
NUM_SUBCORES = 16
NW = NUM_CORES * NUM_SUBCORES
BATCH = 4096
SEQ = 200
BW = BATCH // NW
TW = 2 * HIDDEN
HT = HIDDEN // 8
CBLK = 256
NBLK = 999936 // CBLK
TAIL0 = NBLK * CBLK
TAILW = VOCAB - TAIL0
RSTEPS = -(-NBLK // NW)
RBUF = 2
GBUF = 4
GOUT = SEQ // GBUF
OW = 128
OROWS = SEQ * HIDDEN * BATCH // OW

_mesh = plsc.VectorSubcoreMesh(core_axis_name="c", subcore_axis_name="s")
_params = pltpu.CompilerParams(use_tc_tiling_on_sc=True, needs_layout_passes=False)


@functools.partial(
    pl.kernel,
    out_type=jax.ShapeDtypeStruct((VOCAB, TW), jnp.float32),
    mesh=_mesh,
    scratch_types=[
        pltpu.VMEM((RBUF, HT, 8, CBLK), jnp.float32),
        pltpu.VMEM((RBUF, CBLK, TW), jnp.float32),
        pltpu.VMEM((HIDDEN, TAILW), jnp.float32),
        pltpu.VMEM((TAILW, TW), jnp.float32),
        pltpu.SemaphoreType.DMA,
        pltpu.SemaphoreType.DMA,
        pltpu.SemaphoreType.DMA,
        pltpu.SemaphoreType.DMA,
    ],
    compiler_params=_params,
)
def _repack(wt_hbm, tab_hbm, in_v, out_v, tin_v, tout_v, si0, si1, so0, so1):
    wid = lax.axis_index("s") * NUM_CORES + lax.axis_index("c")
    sin = [si0, si1]
    sout = [so0, so1]
    lane_iota = lax.iota(jnp.int32, LANES)
    ci = [lane_iota + g * LANES for g in range(CBLK // LANES)]
    hvec = [lane_iota + h0 * LANES for h0 in range(HIDDEN // LANES)]
    hdiv = [lax.shift_right_logical(h, 3) for h in hvec]
    hmod = [jnp.bitwise_and(h, 7) for h in hvec]
    scale = jnp.float32(SCALE)

    def in_cps(t, k):
        c0 = (wid + t * NW) * CBLK
        return [
            pltpu.make_async_copy(
                wt_hbm.at[pl.ds(8 * i, 8), pl.ds(c0, CBLK)],
                in_v.at[k, i],
                sin[k],
            )
            for i in range(HT)
        ]

    def out_cp(t, k):
        c0 = (wid + t * NW) * CBLK
        return pltpu.make_async_copy(
            out_v.at[k], tab_hbm.at[pl.ds(c0, CBLK)], sout[k]
        )

    for k in range(RBUF):
        @pl.when(wid + k * NW < NBLK)
        def _():
            for cp in in_cps(k, k):
                cp.start()

    def outer(T, carry):
        for k in range(RBUF):
            t = T * RBUF + k

            @pl.when(wid + t * NW < NBLK)
            def _():
                for cp in in_cps(t, k):
                    cp.wait()

                @pl.when(t >= RBUF)
                def _():
                    out_cp(t - RBUF, k).wait()

                def diag(u0, hc):
                    uv = jnp.bitwise_and(u0 + lane_iota, CBLK - 1)
                    for hi in range(HIDDEN // LANES):
                        vv = plsc.load_gather(
                            in_v.at[k], [hdiv[hi], hmod[hi], uv]
                        )
                        plsc.store_scatter(
                            out_v.at[k], [uv, hvec[hi]], vv * scale
                        )
                    return hc

                lax.fori_loop(0, CBLK, diag, 0, unroll=8)
                out_cp(t, k).start()
                tn = t + RBUF

                @pl.when(wid + tn * NW < NBLK)
                def _():
                    for cp in in_cps(tn, k):
                        cp.start()

        return carry

    lax.fori_loop(0, -(-RSTEPS // RBUF), outer, 0)
    for k in range(RBUF):
        t_last = RSTEPS - RBUF + k

        @pl.when(wid + t_last * NW < NBLK)
        def _():
            out_cp(t_last, k).wait()

    @pl.when(wid == NW - 1)
    def _tail():
        pltpu.sync_copy(wt_hbm.at[pl.ds(0, HIDDEN), pl.ds(TAIL0, TAILW)], tin_v)
        for h in range(HIDDEN):
            hv = jnp.full((LANES,), h, jnp.int32)
            for g in range(TAILW // LANES):
                seg = tin_v[h, pl.ds(g * LANES, LANES)]
                plsc.store_scatter(tout_v, [ci[g], hv], seg * scale)
        pltpu.sync_copy(tout_v, tab_hbm.at[pl.ds(TAIL0, TAILW)])


@functools.partial(
    pl.kernel,
    out_type=jax.ShapeDtypeStruct((OROWS, OW), jnp.float32),
    mesh=_mesh,
    scratch_types=[
        pltpu.VMEM((SEQ, BW), jnp.int32),
        pltpu.VMEM((GBUF, BW, TW), jnp.float32),
        pltpu.VMEM((GBUF, HIDDEN, BW), jnp.float32),
        pltpu.VMEM((GBUF, HIDDEN), jnp.int32),
        pltpu.SemaphoreType.DMA,
        pltpu.SemaphoreType.DMA,
        pltpu.SemaphoreType.DMA,
        pltpu.SemaphoreType.DMA,
        pltpu.SemaphoreType.DMA,
        pltpu.SemaphoreType.DMA,
        pltpu.SemaphoreType.DMA,
        pltpu.SemaphoreType.DMA,
    ],
    compiler_params=_params,
)
def _embed(
    xt_hbm, tab_hbm, out_hbm, idx_v, unit_v, ot_v, sidx_v,
    g0, g1, g2, g3, o0, o1, o2, o3,
):
    wid = lax.axis_index("s") * NUM_CORES + lax.axis_index("c")
    b0 = wid * BW
    gsem = [g0, g1, g2, g3]
    osem = [o0, o1, o2, o3]
    lane_iota = lax.iota(jnp.int32, LANES)
    hvec = [lane_iota + h0 * LANES for h0 in range(HIDDEN // LANES)]
    zeros = jnp.zeros((LANES,), jnp.float32)
    hstep = BATCH // OW

    pltpu.sync_copy(xt_hbm.at[pl.ds(0, SEQ), pl.ds(b0, BW)], idx_v)

    def g_cp(s, k):
        return pltpu.make_async_copy(
            tab_hbm.at[idx_v.at[s]], unit_v.at[k], gsem[k]
        )

    def o_cp(k):
        return pltpu.make_async_copy(
            ot_v.at[k], out_hbm.at[sidx_v.at[k]], osem[k]
        )

    def set_sidx(s, k):
        base = s * (HIDDEN * hstep) + wid
        for g in range(HIDDEN // LANES):
            sidx_v[k, pl.ds(g * LANES, LANES)] = (
                base + (g * LANES + lane_iota) * hstep
            )

    for k in range(GBUF):
        g_cp(k, k).start()

    def outer(T, carry):
        for k in range(GBUF):
            s = T * GBUF + k
            g_cp(s, k).wait()

            @pl.when(T > 0)
            def _():
                o_cp(k).wait()

            set_sidx(s, k)

            def diag(r0, gc):
                rv = jnp.bitwise_and(r0 + lane_iota, BW - 1)
                for hi in range(HIDDEN // LANES):
                    vv = plsc.load_gather(unit_v.at[k], [rv, hvec[hi]])
                    plsc.store_scatter(ot_v.at[k], [hvec[hi], rv], vv)
                return gc

            lax.fori_loop(0, BW, diag, 0, unroll=8)

            def zfix(g, gc):
                iv = idx_v[s, pl.ds(g * LANES, LANES)]
                nz = plsc.all_reduce_population_count(iv == 0)

                @pl.when(nz[0] > 0)
                def _():
                    for j in range(LANES):
                        @pl.when(iv[j] == 0)
                        def _():
                            r = g * LANES + j
                            for c in range(HIDDEN // LANES):
                                plsc.store_scatter(
                                    ot_v.at[k],
                                    [
                                        c * LANES + lane_iota,
                                        jnp.full((LANES,), r, jnp.int32),
                                    ],
                                    zeros,
                                )
                return gc

            lax.fori_loop(0, BW // LANES, zfix, 0)
            o_cp(k).start()

            @pl.when(s + GBUF < SEQ)
            def _():
                g_cp(s + GBUF, k).start()

        return carry

    lax.fori_loop(0, GOUT, outer, 0)
    for k in range(GBUF):
        o_cp(k).wait()


def kernel(x, shared_weights):
    xt = x.astype(jnp.int32).T
    wt = shared_weights.T
    tab = _repack(wt)
    out_r = _embed(xt, tab)
    out_st = out_r.reshape(SEQ, HIDDEN, BATCH)
    return jnp.transpose(out_st, (2, 0, 1))
